# Initial kernel scaffold; baseline (speedup 1.0000x reference)
import jax
import jax.numpy as jnp
from jax.experimental import pallas as pl


def kernel(x, edge_index, W1_self, W1_neigh, b1, Wp2, bp2, W2_self, W2_neigh, b2, W3_neigh, b3, W4_self, W4_neigh, b4, Wfc, bfc):
    raise NotImplementedError("write your pallas kernel here")


# speedup vs baseline: 1.1217x; 1.1217x over previous
import jax
import jax.numpy as jnp
from jax import lax
from jax.experimental import pallas as pl
from jax.experimental.pallas import tpu as pltpu
from jax.experimental.pallas import tpu_sc as plsc

_NW = 32
_GB = 128
_SC_PARAMS = pltpu.CompilerParams(needs_layout_passes=False)


def _make_sc_sum(d, nblk, npad):
  mesh = plsc.VectorSubcoreMesh(core_axis_name="c", subcore_axis_name="s")
  out_type = jax.ShapeDtypeStruct((2, npad, d), jnp.float32)
  cch = 16
  nch = nblk // cch
  scratch = [
      pltpu.VMEM((cch, _GB), jnp.int32),
      pltpu.VMEM((cch, _GB), jnp.int32),
      pltpu.VMEM((_GB, d), jnp.float32),
      pltpu.SemaphoreType.DMA,
      pltpu.VMEM_SHARED((npad, d), jnp.float32),
  ]
  rz = npad // 16

  def body(h_hbm, src_hbm, dst_hbm, z128_hbm,
           out_hbm, idxs, idxd, rows, sem, acc_sh):
    cid = lax.axis_index("c")
    sid = lax.axis_index("s")
    wid = sid * 2 + cid
    pltpu.sync_copy(z128_hbm.at[pl.ds(sid * rz, rz)],
                    acc_sh.at[pl.ds(sid * rz, rz)])
    plsc.subcore_barrier()

    def chunk(ch, carry):
      pltpu.sync_copy(src_hbm.at[wid, pl.ds(ch * cch, cch)], idxs)
      pltpu.sync_copy(dst_hbm.at[wid, pl.ds(ch * cch, cch)], idxd)

      def blk(b, c2):
        pltpu.async_copy(h_hbm.at[idxs.at[b]], rows, sem).wait()
        pltpu.sync_copy(rows, acc_sh.at[idxd.at[b]], add=True)
        return c2

      lax.fori_loop(0, cch, blk, 0)
      return carry

    lax.fori_loop(0, nch, chunk, 0)
    plsc.subcore_barrier()
    pltpu.sync_copy(acc_sh.at[pl.ds(sid * rz, rz)],
                    out_hbm.at[cid, pl.ds(sid * rz, rz)])

  return pl.kernel(body, out_type=out_type, mesh=mesh, scratch_types=scratch,
                   compiler_params=_SC_PARAMS)


def _make_sc_deg(nblk_tot, npt):
  mesh = plsc.VectorSubcoreMesh(core_axis_name="c", subcore_axis_name="s")
  cch = 16
  nch = nblk_tot // cch
  drows = npt // 8

  out_type = jax.ShapeDtypeStruct((_NW * drows, 128), jnp.float32)
  scratch = [
      pltpu.VMEM((cch, _GB), jnp.int32),
      pltpu.VMEM((drows, 128), jnp.float32),
  ]

  def body(dst2_hbm, out_hbm, dbuf, dacc):
    cid = lax.axis_index("c")
    sid = lax.axis_index("s")
    wid = sid * 2 + cid
    lo = wid * npt
    zeros = jnp.zeros((16,), jnp.float32)

    def zrow(r, carry):
      for c in range(8):
        dacc[r, pl.ds(c * 16, 16)] = zeros
      return carry

    lax.fori_loop(0, drows, zrow, 0)

    lane = lax.iota(jnp.int32, 16)
    onesv = jnp.ones((16,), jnp.float32)

    def chunk(ch, carry):
      pltpu.sync_copy(dst2_hbm.at[pl.ds(ch * cch, cch)], dbuf)

      def row(r, c2):
        for c in range(_GB // 16):
          dv = dbuf[r, pl.ds(c * 16, 16)] - lo
          msk = (dv >= 0) & (dv < npt)
          rr = lax.shift_right_logical(dv, 3)
          cc = lax.shift_left(dv & 7, 4) + lane
          plsc.addupdate_scatter(dacc, [rr, cc], onesv, mask=msk)
        return c2

      lax.fori_loop(0, cch, row, 0)
      return carry

    lax.fori_loop(0, nch, chunk, 0)
    pltpu.sync_copy(dacc, out_hbm.at[pl.ds(wid * drows, drows)])

  return pl.kernel(body, out_type=out_type, mesh=mesh, scratch_types=scratch,
                   compiler_params=_SC_PARAMS)


def _make_sc_max(d, nblk_tot, npt):
  mesh = plsc.VectorSubcoreMesh(core_axis_name="c", subcore_axis_name="s")
  cch = 16
  nch = nblk_tot // cch
  cap = cch * _GB + 32
  acc_rows = npt + 8

  out_type = jax.ShapeDtypeStruct((_NW * npt, d), jnp.float32)
  scratch = [
      pltpu.VMEM((cch, _GB), jnp.int32),
      pltpu.VMEM((cch, _GB), jnp.int32),
      pltpu.VMEM((cap,), jnp.int32),
      pltpu.VMEM((cap,), jnp.int32),
      pltpu.VMEM((32, d), jnp.float32),
      pltpu.VMEM((acc_rows, d), jnp.float32),
      pltpu.SemaphoreType.DMA,
  ]

  def body(m_hbm, src2_hbm, dst2_hbm, z128_hbm,
           out_hbm, sbuf, dbuf, selsrc, seldst, rows2, acc, sem):
    cid = lax.axis_index("c")
    sid = lax.axis_index("s")
    wid = sid * 2 + cid
    lo = wid * npt
    pltpu.sync_copy(z128_hbm.at[pl.ds(0, acc_rows)], acc)

    lane = lax.iota(jnp.int32, 16)
    pad_src = jnp.zeros((16,), jnp.int32)
    pad_dst = jnp.full((16,), npt, jnp.int32)

    def chunk(ch, carry):
      pltpu.sync_copy(src2_hbm.at[pl.ds(ch * cch, cch)], sbuf)
      pltpu.sync_copy(dst2_hbm.at[pl.ds(ch * cch, cch)], dbuf)

      def row(r, nsel):
        for c in range(_GB // 16):
          dv = dbuf[r, pl.ds(c * 16, 16)]
          sv = sbuf[r, pl.ds(c * 16, 16)]
          msk = (dv >= lo) & (dv < lo + npt)
          inc = plsc.cumsum(msk.astype(jnp.int32))
          pos = nsel + inc - 1
          plsc.store_scatter(selsrc, [pos], sv, mask=msk)
          plsc.store_scatter(seldst, [pos], dv - lo, mask=msk)
          nsel = nsel + inc[15]
        return nsel

      nsel = lax.fori_loop(0, cch, row, jnp.int32(0))
      plsc.store_scatter(selsrc, [nsel + lane], pad_src)
      plsc.store_scatter(seldst, [nsel + lane], pad_dst)
      plsc.store_scatter(selsrc, [nsel + 16 + lane], pad_src)
      plsc.store_scatter(seldst, [nsel + 16 + lane], pad_dst)
      nb = lax.shift_right_logical(nsel + 31, 5)

      def gblk(b, c2):
        pltpu.async_copy(m_hbm.at[selsrc.at[pl.ds(b * 32, 32)]], rows2,
                         sem).wait()
        for k in range(2):
          dvec = seldst[pl.ds(b * 32 + k * 16, 16)]
          for j in range(16):
            dl = dvec[j]
            for c in range(d // 16):
              cur = acc[dl, pl.ds(c * 16, 16)]
              val = rows2[k * 16 + j, pl.ds(c * 16, 16)]
              acc[dl, pl.ds(c * 16, 16)] = jnp.maximum(cur, val)
        return c2

      lax.fori_loop(0, nb, gblk, 0)
      return carry

    lax.fori_loop(0, nch, chunk, 0)
    pltpu.sync_copy(acc.at[pl.ds(0, npt)], out_hbm.at[pl.ds(lo, npt)])

  return pl.kernel(body, out_type=out_type, mesh=mesh, scratch_types=scratch,
                   compiler_params=_SC_PARAMS)


def _tc_call(fn, n, blk, args, n_out):
  grid = (n // blk,)
  in_specs = []
  for a in args:
    if a.ndim == 3:
      in_specs.append(pl.BlockSpec((2, blk, a.shape[2]),
                                   lambda i: (0, i, 0)))
    elif a.shape[0] >= n:
      in_specs.append(pl.BlockSpec((blk, a.shape[1]), lambda i: (i, 0)))
    else:
      in_specs.append(pl.BlockSpec(a.shape, lambda i: (0,) * a.ndim))
  d = args[0].shape[-1]
  out_specs = [pl.BlockSpec((blk, d), lambda i: (i, 0))] * n_out
  out_shape = [jax.ShapeDtypeStruct((n, d), jnp.float32)] * n_out
  if n_out == 1:
    out_specs, out_shape = out_specs[0], out_shape[0]
  return pl.pallas_call(fn, grid=grid, in_specs=in_specs,
                        out_specs=out_specs, out_shape=out_shape)(*args)


def _deg_of(dg_blk):
  return jnp.sum(dg_blk, axis=1, keepdims=True)


def _tc1(x_r, a_r, dg_r, w1s_r, w1n_r, b1_r, wp2_r, bp2_r, h1_r, m_r):
  deg = _deg_of(dg_r[...])
  agg = (a_r[0] + a_r[1]) / jnp.maximum(deg, 1.0)
  h1 = (jnp.dot(x_r[...], w1s_r[...], preferred_element_type=jnp.float32)
        + jnp.dot(agg, w1n_r[...], preferred_element_type=jnp.float32)
        + b1_r[...])
  h1_r[...] = h1
  m_r[...] = jnp.maximum(
      jnp.dot(h1, wp2_r[...], preferred_element_type=jnp.float32)
      + bp2_r[...], 0.0)


def _tc2(h1_r, p_r, w2s_r, w2n_r, b2_r, h2_r):
  h2_r[...] = (jnp.dot(h1_r[...], w2s_r[...],
                       preferred_element_type=jnp.float32)
               + jnp.dot(p_r[...], w2n_r[...],
                         preferred_element_type=jnp.float32)
               + b2_r[...])


def _tc3(h2_r, s_r, dg_r, w3n_r, b3_r, h3_r):
  deg = _deg_of(dg_r[...])
  hg = (s_r[0] + s_r[1] + h2_r[...]) / (deg + 1.0)
  h3_r[...] = (jnp.dot(hg, w3n_r[...], preferred_element_type=jnp.float32)
               + b3_r[...])


def _tc4(h3_r, a_r, dg_r, w4s_r, w4n_r, b4_r, wfc_r, bfc_r, o_r):
  deg = _deg_of(dg_r[...])
  agg = (a_r[0] + a_r[1]) / jnp.maximum(deg, 1.0)
  h4 = (jnp.dot(h3_r[...], w4s_r[...], preferred_element_type=jnp.float32)
        + jnp.dot(agg, w4n_r[...], preferred_element_type=jnp.float32)
        + b4_r[...])
  o_r[...] = (jnp.dot(h4, wfc_r[...], preferred_element_type=jnp.float32)
              + bfc_r[...])


def kernel(x, edge_index, W1_self, W1_neigh, b1, Wp2, bp2, W2_self, W2_neigh,
           b2, W3_neigh, b3, W4_self, W4_neigh, b4, Wfc, bfc):
  n, d = x.shape
  e = edge_index.shape[1]
  ept = e // _NW
  nblk = -(-(-(-ept // _GB)) // 16) * 16
  padt = nblk * _GB
  pad = padt - ept

  src = edge_index[0].reshape(_NW, ept)
  dst = edge_index[1].reshape(_NW, ept)
  src3 = jnp.pad(src, ((0, 0), (0, pad))).reshape(_NW, nblk, _GB)
  dst3 = jnp.pad(dst, ((0, 0), (0, pad)),
                 constant_values=n).reshape(_NW, nblk, _GB)
  src2 = src3.reshape(_NW * nblk, _GB)
  dst2 = dst3.reshape(_NW * nblk, _GB)

  npad = -(-(n + 1) // 128) * 128
  npt = -(-(n + 1) // (_NW * 8)) * 8
  zeros128 = jnp.zeros((npad, d), jnp.float32)

  b1r = b1.reshape(1, d)
  bp2r = bp2.reshape(1, d)
  b2r = b2.reshape(1, d)
  b3r = b3.reshape(1, d)
  b4r = b4.reshape(1, d)
  bfcr = bfc.reshape(1, d)

  sum_pass = _make_sc_sum(d, nblk, npad)
  deg_pass = _make_sc_deg(_NW * nblk, npt)
  seg_max = _make_sc_max(d, _NW * nblk, npt)

  blk = 2000

  degp = deg_pass(dst2).reshape(_NW * npt, 16)
  a1 = sum_pass(x, src3, dst3, zeros128)
  h1, m = _tc_call(_tc1, n, blk,
                   (x, a1, degp, W1_self, W1_neigh, b1r, Wp2, bp2r), 2)
  pooled = seg_max(m, src2, dst2, zeros128)
  h2 = _tc_call(_tc2, n, blk, (h1, pooled, W2_self, W2_neigh, b2r), 1)
  s3 = sum_pass(h2, src3, dst3, zeros128)
  h3 = _tc_call(_tc3, n, blk, (h2, s3, degp, W3_neigh, b3r), 1)
  a4 = sum_pass(h3, src3, dst3, zeros128)
  out = _tc_call(_tc4, n, blk,
                 (h3, a4, degp, W4_self, W4_neigh, b4r, Wfc, bfcr), 1)
  return out


# speedup vs baseline: 2.3183x; 2.0668x over previous
import jax
import jax.numpy as jnp
from jax import lax
from jax.experimental import pallas as pl
from jax.experimental.pallas import tpu as pltpu
from jax.experimental.pallas import tpu_sc as plsc

_NW = 32
_GB = 128
_SC_PARAMS = pltpu.CompilerParams(needs_layout_passes=False)


def _make_sc_sum(d, nblk, npad):
  mesh = plsc.VectorSubcoreMesh(core_axis_name="c", subcore_axis_name="s")
  out_type = jax.ShapeDtypeStruct((2, npad, d), jnp.float32)
  cch = 16
  nch = nblk // cch
  scratch = [
      pltpu.VMEM((cch, _GB), jnp.int32),
      pltpu.VMEM((cch, _GB), jnp.int32),
      pltpu.VMEM((2, _GB, d), jnp.float32),
      pltpu.SemaphoreType.DMA,
      pltpu.SemaphoreType.DMA,
      pltpu.VMEM_SHARED((npad, d), jnp.float32),
  ]
  rz = npad // 16

  def body(h_hbm, src_hbm, dst_hbm, z128_hbm,
           out_hbm, idxs, idxd, rows, g0, g1, acc_sh):
    cid = lax.axis_index("c")
    sid = lax.axis_index("s")
    wid = sid * 2 + cid
    pltpu.sync_copy(z128_hbm.at[pl.ds(sid * rz, rz)],
                    acc_sh.at[pl.ds(sid * rz, rz)])
    plsc.subcore_barrier()

    def chunk(ch, carry):
      pltpu.sync_copy(src_hbm.at[wid, pl.ds(ch * cch, cch)], idxs)
      pltpu.sync_copy(dst_hbm.at[wid, pl.ds(ch * cch, cch)], idxd)
      pltpu.async_copy(h_hbm.at[idxs.at[0]], rows.at[0], g0)

      def blk2(i, c2):
        b0 = i * 2
        pltpu.make_async_copy(h_hbm.at[idxs.at[b0]], rows.at[0], g0).wait()
        pltpu.async_copy(h_hbm.at[idxs.at[b0 + 1]], rows.at[1], g1)
        pltpu.sync_copy(rows.at[0], acc_sh.at[idxd.at[b0]], add=True)
        pltpu.make_async_copy(h_hbm.at[idxs.at[b0 + 1]], rows.at[1],
                              g1).wait()

        @pl.when(i + 1 < cch // 2)
        def _():
          pltpu.async_copy(h_hbm.at[idxs.at[b0 + 2]], rows.at[0], g0)

        pltpu.sync_copy(rows.at[1], acc_sh.at[idxd.at[b0 + 1]], add=True)
        return c2

      lax.fori_loop(0, cch // 2, blk2, 0)
      return carry

    lax.fori_loop(0, nch, chunk, 0)
    plsc.subcore_barrier()
    pltpu.sync_copy(acc_sh.at[pl.ds(sid * rz, rz)],
                    out_hbm.at[cid, pl.ds(sid * rz, rz)])

  return pl.kernel(body, out_type=out_type, mesh=mesh, scratch_types=scratch,
                   compiler_params=_SC_PARAMS)


def _make_sc_deg(nblk_tot, npt):
  mesh = plsc.VectorSubcoreMesh(core_axis_name="c", subcore_axis_name="s")
  cch = 32
  nch = nblk_tot // cch
  drows = npt // 8

  out_type = jax.ShapeDtypeStruct((_NW * drows, 128), jnp.float32)
  scratch = [
      pltpu.VMEM((2, cch, _GB), jnp.int32),
      pltpu.VMEM((drows, 128), jnp.float32),
      pltpu.SemaphoreType.DMA,
  ]

  def body(dst2_hbm, out_hbm, dbuf, dacc, st):
    cid = lax.axis_index("c")
    sid = lax.axis_index("s")
    wid = sid * 2 + cid
    lo = wid * npt
    zeros = jnp.zeros((16,), jnp.float32)

    def zrow(r, carry):
      for c in range(8):
        dacc[r, pl.ds(c * 16, 16)] = zeros
      return carry

    lax.fori_loop(0, drows, zrow, 0)

    lane = lax.iota(jnp.int32, 16)
    onesv = jnp.ones((16,), jnp.float32)
    pltpu.sync_copy(dst2_hbm.at[pl.ds(0, cch)], dbuf.at[0])

    def chunk(ch, carry):
      slot = ch % 2
      nxt = (ch + 1) % 2

      @pl.when(ch + 1 < nch)
      def _():
        pltpu.async_copy(dst2_hbm.at[pl.ds((ch + 1) * cch, cch)],
                         dbuf.at[nxt], st)

      def row(r, c2):
        for c in range(_GB // 16):
          dv = dbuf[slot, r, pl.ds(c * 16, 16)] - lo
          msk = (dv >= 0) & (dv < npt)
          rr = lax.shift_right_logical(dv, 3)
          cc = lax.shift_left(dv & 7, 4) + lane
          plsc.addupdate_scatter(dacc, [rr, cc], onesv, mask=msk)
        return c2

      lax.fori_loop(0, cch, row, 0)

      @pl.when(ch + 1 < nch)
      def _():
        pltpu.make_async_copy(dst2_hbm.at[pl.ds((ch + 1) * cch, cch)],
                              dbuf.at[nxt], st).wait()

      return carry

    lax.fori_loop(0, nch, chunk, 0)
    pltpu.sync_copy(dacc, out_hbm.at[pl.ds(wid * drows, drows)])

  return pl.kernel(body, out_type=out_type, mesh=mesh, scratch_types=scratch,
                   compiler_params=_SC_PARAMS)


def _make_sc_max(d, nblk_tot, npt):
  mesh = plsc.VectorSubcoreMesh(core_axis_name="c", subcore_axis_name="s")
  cch = 32
  nch = nblk_tot // cch
  cap = 8192
  acc_rows = npt + 8

  out_type = jax.ShapeDtypeStruct((_NW * npt, d), jnp.float32)
  scratch = [
      pltpu.VMEM((2, cch, _GB), jnp.int32),
      pltpu.VMEM((2, cch, _GB), jnp.int32),
      pltpu.VMEM((cap,), jnp.int32),
      pltpu.VMEM((cap,), jnp.int32),
      pltpu.VMEM((_GB, d), jnp.float32),
      pltpu.VMEM((acc_rows, d), jnp.float32),
      pltpu.SemaphoreType.DMA,
      pltpu.SemaphoreType.DMA,
  ]

  def body(m_hbm, src2_hbm, dst2_hbm, z128_hbm,
           out_hbm, sbuf, dbuf, selsrc, seldst, rows2, acc, st, gs):
    cid = lax.axis_index("c")
    sid = lax.axis_index("s")
    wid = sid * 2 + cid
    lo = wid * npt
    pltpu.sync_copy(z128_hbm.at[pl.ds(0, acc_rows)], acc)

    lane = lax.iota(jnp.int32, 16)
    pad_src = jnp.zeros((16,), jnp.int32)
    pad_dst = jnp.full((16,), npt, jnp.int32)
    capm = cap - 1

    def flush(nb, flushed):
      def gblk(b, c2):
        off = pl.multiple_of((flushed + b * _GB) & capm, _GB)
        pltpu.async_copy(m_hbm.at[selsrc.at[pl.ds(off, _GB)]], rows2,
                         gs).wait()

        def grp(k, c3):
          dvec = seldst[pl.ds(pl.multiple_of(off + k * 16, 16), 16)]
          for j in range(16):
            dl = dvec[j]
            for c in range(d // 16):
              cur = acc[dl, pl.ds(c * 16, 16)]
              val = rows2[k * 16 + j, pl.ds(c * 16, 16)]
              acc[dl, pl.ds(c * 16, 16)] = jnp.maximum(cur, val)
          return c3

        lax.fori_loop(0, _GB // 16, grp, 0)
        return c2

      lax.fori_loop(0, nb, gblk, 0)
      return flushed + nb * _GB

    pltpu.sync_copy(src2_hbm.at[pl.ds(0, cch)], sbuf.at[0])
    pltpu.sync_copy(dst2_hbm.at[pl.ds(0, cch)], dbuf.at[0])

    def chunk(ch, carry):
      nsel, flushed = carry
      slot = ch % 2
      nxt = (ch + 1) % 2

      @pl.when(ch + 1 < nch)
      def _():
        pltpu.async_copy(src2_hbm.at[pl.ds((ch + 1) * cch, cch)],
                         sbuf.at[nxt], st)
        pltpu.async_copy(dst2_hbm.at[pl.ds((ch + 1) * cch, cch)],
                         dbuf.at[nxt], st)

      def row(r, nsel):
        for c in range(_GB // 16):
          dv = dbuf[slot, r, pl.ds(c * 16, 16)]
          sv = sbuf[slot, r, pl.ds(c * 16, 16)]
          msk = (dv >= lo) & (dv < lo + npt)
          inc = plsc.cumsum(msk.astype(jnp.int32))
          pos = (nsel + inc - 1) & capm
          plsc.store_scatter(selsrc, [pos], sv, mask=msk)
          plsc.store_scatter(seldst, [pos], dv - lo, mask=msk)
          nsel = nsel + inc[15]
        return nsel

      nsel = lax.fori_loop(0, cch, row, nsel)
      nb = lax.shift_right_logical(nsel - flushed, 7)
      flushed = flush(nb, flushed)

      @pl.when(ch + 1 < nch)
      def _():
        pltpu.make_async_copy(src2_hbm.at[pl.ds((ch + 1) * cch, cch)],
                              sbuf.at[nxt], st).wait()
        pltpu.make_async_copy(dst2_hbm.at[pl.ds((ch + 1) * cch, cch)],
                              dbuf.at[nxt], st).wait()

      return (nsel, flushed)

    nsel, flushed = lax.fori_loop(0, nch, chunk,
                                  (jnp.int32(0), jnp.int32(0)))
    for k in range(_GB // 16):
      plsc.store_scatter(selsrc, [(nsel + k * 16 + lane) & capm], pad_src)
      plsc.store_scatter(seldst, [(nsel + k * 16 + lane) & capm], pad_dst)
    nb2 = lax.shift_right_logical(nsel - flushed + _GB - 1, 7)
    flush(nb2, flushed)
    pltpu.sync_copy(acc.at[pl.ds(0, npt)], out_hbm.at[pl.ds(lo, npt)])

  return pl.kernel(body, out_type=out_type, mesh=mesh, scratch_types=scratch,
                   compiler_params=_SC_PARAMS)


def _tc_call(fn, n, blk, args, n_out):
  grid = (n // blk,)
  in_specs = []
  for a in args:
    if a.ndim == 3:
      in_specs.append(pl.BlockSpec((2, blk, a.shape[2]),
                                   lambda i: (0, i, 0)))
    elif a.shape[0] >= n:
      in_specs.append(pl.BlockSpec((blk, a.shape[1]), lambda i: (i, 0)))
    else:
      in_specs.append(pl.BlockSpec(a.shape, lambda i: (0,) * a.ndim))
  d = args[0].shape[-1]
  out_specs = [pl.BlockSpec((blk, d), lambda i: (i, 0))] * n_out
  out_shape = [jax.ShapeDtypeStruct((n, d), jnp.float32)] * n_out
  if n_out == 1:
    out_specs, out_shape = out_specs[0], out_shape[0]
  return pl.pallas_call(fn, grid=grid, in_specs=in_specs,
                        out_specs=out_specs, out_shape=out_shape)(*args)


def _deg_of(dg_blk):
  return jnp.sum(dg_blk, axis=1, keepdims=True)


def _tc1(x_r, a_r, dg_r, w1s_r, w1n_r, b1_r, wp2_r, bp2_r, h1_r, m_r):
  deg = _deg_of(dg_r[...])
  agg = (a_r[0] + a_r[1]) / jnp.maximum(deg, 1.0)
  h1 = (jnp.dot(x_r[...], w1s_r[...], preferred_element_type=jnp.float32)
        + jnp.dot(agg, w1n_r[...], preferred_element_type=jnp.float32)
        + b1_r[...])
  h1_r[...] = h1
  m_r[...] = jnp.maximum(
      jnp.dot(h1, wp2_r[...], preferred_element_type=jnp.float32)
      + bp2_r[...], 0.0)


def _tc2(h1_r, p_r, w2s_r, w2n_r, b2_r, h2_r):
  h2_r[...] = (jnp.dot(h1_r[...], w2s_r[...],
                       preferred_element_type=jnp.float32)
               + jnp.dot(p_r[...], w2n_r[...],
                         preferred_element_type=jnp.float32)
               + b2_r[...])


def _tc3(h2_r, s_r, dg_r, w3n_r, b3_r, h3_r):
  deg = _deg_of(dg_r[...])
  hg = (s_r[0] + s_r[1] + h2_r[...]) / (deg + 1.0)
  h3_r[...] = (jnp.dot(hg, w3n_r[...], preferred_element_type=jnp.float32)
               + b3_r[...])


def _tc4(h3_r, a_r, dg_r, w4s_r, w4n_r, b4_r, wfc_r, bfc_r, o_r):
  deg = _deg_of(dg_r[...])
  agg = (a_r[0] + a_r[1]) / jnp.maximum(deg, 1.0)
  h4 = (jnp.dot(h3_r[...], w4s_r[...], preferred_element_type=jnp.float32)
        + jnp.dot(agg, w4n_r[...], preferred_element_type=jnp.float32)
        + b4_r[...])
  o_r[...] = (jnp.dot(h4, wfc_r[...], preferred_element_type=jnp.float32)
              + bfc_r[...])


def kernel(x, edge_index, W1_self, W1_neigh, b1, Wp2, bp2, W2_self, W2_neigh,
           b2, W3_neigh, b3, W4_self, W4_neigh, b4, Wfc, bfc):
  n, d = x.shape
  e = edge_index.shape[1]
  ept = e // _NW
  nblk = -(-(-(-ept // _GB)) // 16) * 16
  padt = nblk * _GB
  pad = padt - ept

  src = edge_index[0].reshape(_NW, ept)
  dst = edge_index[1].reshape(_NW, ept)
  src3 = jnp.pad(src, ((0, 0), (0, pad))).reshape(_NW, nblk, _GB)
  dst3 = jnp.pad(dst, ((0, 0), (0, pad)),
                 constant_values=n).reshape(_NW, nblk, _GB)
  src2 = src3.reshape(_NW * nblk, _GB)
  dst2 = dst3.reshape(_NW * nblk, _GB)

  npad = -(-(n + 1) // 128) * 128
  npt = -(-(n + 1) // (_NW * 8)) * 8
  zeros128 = jnp.zeros((npad, d), jnp.float32)

  b1r = b1.reshape(1, d)
  bp2r = bp2.reshape(1, d)
  b2r = b2.reshape(1, d)
  b3r = b3.reshape(1, d)
  b4r = b4.reshape(1, d)
  bfcr = bfc.reshape(1, d)

  sum_pass = _make_sc_sum(d, nblk, npad)
  deg_pass = _make_sc_deg(_NW * nblk, npt)
  seg_max = _make_sc_max(d, _NW * nblk, npt)

  blk = 2000

  degp = deg_pass(dst2).reshape(_NW * npt, 16)
  a1 = sum_pass(x, src3, dst3, zeros128)
  h1, m = _tc_call(_tc1, n, blk,
                   (x, a1, degp, W1_self, W1_neigh, b1r, Wp2, bp2r), 2)
  pooled = seg_max(m, src2, dst2, zeros128)
  h2 = _tc_call(_tc2, n, blk, (h1, pooled, W2_self, W2_neigh, b2r), 1)
  s3 = sum_pass(h2, src3, dst3, zeros128)
  h3 = _tc_call(_tc3, n, blk, (h2, s3, degp, W3_neigh, b3r), 1)
  a4 = sum_pass(h3, src3, dst3, zeros128)
  out = _tc_call(_tc4, n, blk,
                 (h3, a4, degp, W4_self, W4_neigh, b4r, Wfc, bfcr), 1)
  return out


# speedup vs baseline: 2.4750x; 1.0676x over previous
import jax
import jax.numpy as jnp
from jax import lax
from jax.experimental import pallas as pl
from jax.experimental.pallas import tpu as pltpu
from jax.experimental.pallas import tpu_sc as plsc

_NW = 32
_GB = 128
_SC_PARAMS = pltpu.CompilerParams(needs_layout_passes=False)


def _make_sc_sum(d, nblk, npad):
  mesh = plsc.VectorSubcoreMesh(core_axis_name="c", subcore_axis_name="s")
  out_type = jax.ShapeDtypeStruct((2, npad, d), jnp.float32)
  cch = 16
  nch = nblk // cch
  scratch = [
      pltpu.VMEM((cch, _GB), jnp.int32),
      pltpu.VMEM((cch, _GB), jnp.int32),
      pltpu.VMEM((2, _GB, d), jnp.float32),
      pltpu.SemaphoreType.DMA,
      pltpu.SemaphoreType.DMA,
      pltpu.VMEM_SHARED((npad, d), jnp.float32),
  ]
  rz = npad // 16

  def body(h_hbm, src_hbm, dst_hbm, z128_hbm,
           out_hbm, idxs, idxd, rows, g0, g1, acc_sh):
    cid = lax.axis_index("c")
    sid = lax.axis_index("s")
    wid = sid * 2 + cid
    pltpu.sync_copy(z128_hbm.at[pl.ds(sid * rz, rz)],
                    acc_sh.at[pl.ds(sid * rz, rz)])
    plsc.subcore_barrier()

    def chunk(ch, carry):
      pltpu.sync_copy(src_hbm.at[wid, pl.ds(ch * cch, cch)], idxs)
      pltpu.sync_copy(dst_hbm.at[wid, pl.ds(ch * cch, cch)], idxd)
      pltpu.async_copy(h_hbm.at[idxs.at[0]], rows.at[0], g0)

      def blk2(i, c2):
        b0 = i * 2
        pltpu.make_async_copy(h_hbm.at[idxs.at[b0]], rows.at[0], g0).wait()
        pltpu.async_copy(h_hbm.at[idxs.at[b0 + 1]], rows.at[1], g1)
        pltpu.sync_copy(rows.at[0], acc_sh.at[idxd.at[b0]], add=True)
        pltpu.make_async_copy(h_hbm.at[idxs.at[b0 + 1]], rows.at[1],
                              g1).wait()

        @pl.when(i + 1 < cch // 2)
        def _():
          pltpu.async_copy(h_hbm.at[idxs.at[b0 + 2]], rows.at[0], g0)

        pltpu.sync_copy(rows.at[1], acc_sh.at[idxd.at[b0 + 1]], add=True)
        return c2

      lax.fori_loop(0, cch // 2, blk2, 0)
      return carry

    lax.fori_loop(0, nch, chunk, 0)
    plsc.subcore_barrier()
    pltpu.sync_copy(acc_sh.at[pl.ds(sid * rz, rz)],
                    out_hbm.at[cid, pl.ds(sid * rz, rz)])

  return pl.kernel(body, out_type=out_type, mesh=mesh, scratch_types=scratch,
                   compiler_params=_SC_PARAMS)


def _make_sc_deg(nblk_tot, npt):
  mesh = plsc.VectorSubcoreMesh(core_axis_name="c", subcore_axis_name="s")
  cch = 32
  nch = nblk_tot // cch
  drows = npt // 8

  out_type = jax.ShapeDtypeStruct((_NW * drows, 128), jnp.float32)
  scratch = [
      pltpu.VMEM((2, cch, _GB), jnp.int32),
      pltpu.VMEM((drows, 128), jnp.float32),
      pltpu.SemaphoreType.DMA,
  ]

  def body(dst2_hbm, out_hbm, dbuf, dacc, st):
    cid = lax.axis_index("c")
    sid = lax.axis_index("s")
    wid = sid * 2 + cid
    lo = wid * npt
    zeros = jnp.zeros((16,), jnp.float32)

    def zrow(r, carry):
      for c in range(8):
        dacc[r, pl.ds(c * 16, 16)] = zeros
      return carry

    lax.fori_loop(0, drows, zrow, 0)

    lane = lax.iota(jnp.int32, 16)
    onesv = jnp.ones((16,), jnp.float32)
    pltpu.sync_copy(dst2_hbm.at[pl.ds(0, cch)], dbuf.at[0])

    def chunk(ch, carry):
      slot = ch % 2
      nxt = (ch + 1) % 2

      @pl.when(ch + 1 < nch)
      def _():
        pltpu.async_copy(dst2_hbm.at[pl.ds((ch + 1) * cch, cch)],
                         dbuf.at[nxt], st)

      def row(r, c2):
        for c in range(_GB // 16):
          dv = dbuf[slot, r, pl.ds(c * 16, 16)] - lo
          msk = (dv >= 0) & (dv < npt)
          rr = lax.shift_right_logical(dv, 3)
          cc = lax.shift_left(dv & 7, 4) + lane
          plsc.addupdate_scatter(dacc, [rr, cc], onesv, mask=msk)
        return c2

      lax.fori_loop(0, cch, row, 0)

      @pl.when(ch + 1 < nch)
      def _():
        pltpu.make_async_copy(dst2_hbm.at[pl.ds((ch + 1) * cch, cch)],
                              dbuf.at[nxt], st).wait()

      return carry

    lax.fori_loop(0, nch, chunk, 0)
    pltpu.sync_copy(dacc, out_hbm.at[pl.ds(wid * drows, drows)])

  return pl.kernel(body, out_type=out_type, mesh=mesh, scratch_types=scratch,
                   compiler_params=_SC_PARAMS)


def _make_sc_max(d, nblk_tot, npt):
  mesh = plsc.VectorSubcoreMesh(core_axis_name="c", subcore_axis_name="s")
  cch = 32
  nch = nblk_tot // cch
  cap = 8192
  acc_rows = npt + 8

  out_type = jax.ShapeDtypeStruct((_NW * npt, d), jnp.float32)
  scratch = [
      pltpu.VMEM((2, cch, _GB), jnp.int32),
      pltpu.VMEM((2, cch, _GB), jnp.int32),
      pltpu.VMEM((cap,), jnp.int32),
      pltpu.VMEM((cap,), jnp.int32),
      pltpu.VMEM((2, _GB, d), jnp.float32),
      pltpu.VMEM((acc_rows, d), jnp.float32),
      pltpu.SemaphoreType.DMA,
      pltpu.SemaphoreType.DMA,
      pltpu.SemaphoreType.DMA,
  ]

  def body(m_hbm, src2_hbm, dst2_hbm, z128_hbm,
           out_hbm, sbuf, dbuf, selsrc, seldst, rows2, acc, st, g0, g1):
    cid = lax.axis_index("c")
    sid = lax.axis_index("s")
    wid = sid * 2 + cid
    lo = wid * npt
    pltpu.sync_copy(z128_hbm.at[pl.ds(0, acc_rows)], acc)

    lane = lax.iota(jnp.int32, 16)
    pad_src = jnp.zeros((16,), jnp.int32)
    pad_dst = jnp.full((16,), npt, jnp.int32)
    capm = cap - 1

    gsem = (g0, g1)

    def flush(nb, flushed):
      def boff(b):
        return pl.multiple_of((flushed + b * _GB) & capm, _GB)

      @pl.when(nb > 0)
      def _():
        pltpu.async_copy(m_hbm.at[selsrc.at[pl.ds(boff(0), _GB)]],
                         rows2.at[0], g0)

      def gblk(b, c2):
        off = boff(b)

        def proc(par):
          pltpu.make_async_copy(m_hbm.at[selsrc.at[pl.ds(off, _GB)]],
                                rows2.at[par], gsem[par]).wait()

          @pl.when(b + 1 < nb)
          def _():
            pltpu.async_copy(m_hbm.at[selsrc.at[pl.ds(boff(b + 1), _GB)]],
                             rows2.at[1 - par], gsem[1 - par])

          def grp(k, c3):
            dvec = seldst[pl.ds(pl.multiple_of(off + k * 16, 16), 16)]
            for j in range(16):
              dl = dvec[j]
              for c in range(d // 16):
                cur = acc[dl, pl.ds(c * 16, 16)]
                val = rows2[par, k * 16 + j, pl.ds(c * 16, 16)]
                acc[dl, pl.ds(c * 16, 16)] = jnp.maximum(cur, val)
            return c3

          lax.fori_loop(0, _GB // 16, grp, 0)

        @pl.when(b % 2 == 0)
        def _():
          proc(0)

        @pl.when(b % 2 == 1)
        def _():
          proc(1)

        return c2

      lax.fori_loop(0, nb, gblk, 0)
      return flushed + nb * _GB

    pltpu.sync_copy(src2_hbm.at[pl.ds(0, cch)], sbuf.at[0])
    pltpu.sync_copy(dst2_hbm.at[pl.ds(0, cch)], dbuf.at[0])

    def chunk(ch, carry):
      nsel, flushed = carry
      slot = ch % 2
      nxt = (ch + 1) % 2

      @pl.when(ch + 1 < nch)
      def _():
        pltpu.async_copy(src2_hbm.at[pl.ds((ch + 1) * cch, cch)],
                         sbuf.at[nxt], st)
        pltpu.async_copy(dst2_hbm.at[pl.ds((ch + 1) * cch, cch)],
                         dbuf.at[nxt], st)

      def row(r, nsel):
        for c in range(_GB // 16):
          dv = dbuf[slot, r, pl.ds(c * 16, 16)]
          sv = sbuf[slot, r, pl.ds(c * 16, 16)]
          msk = (dv >= lo) & (dv < lo + npt)
          inc = plsc.cumsum(msk.astype(jnp.int32))
          pos = (nsel + inc - 1) & capm
          plsc.store_scatter(selsrc, [pos], sv, mask=msk)
          plsc.store_scatter(seldst, [pos], dv - lo, mask=msk)
          nsel = nsel + inc[15]
        return nsel

      nsel = lax.fori_loop(0, cch, row, nsel)
      nbavail = lax.shift_right_logical(nsel - flushed, 7)
      flushed = flush(nbavail - (nbavail & 1), flushed)

      @pl.when(ch + 1 < nch)
      def _():
        pltpu.make_async_copy(src2_hbm.at[pl.ds((ch + 1) * cch, cch)],
                              sbuf.at[nxt], st).wait()
        pltpu.make_async_copy(dst2_hbm.at[pl.ds((ch + 1) * cch, cch)],
                              dbuf.at[nxt], st).wait()

      return (nsel, flushed)

    nsel, flushed = lax.fori_loop(0, nch, chunk,
                                  (jnp.int32(0), jnp.int32(0)))
    for k in range(_GB // 16):
      plsc.store_scatter(selsrc, [(nsel + k * 16 + lane) & capm], pad_src)
      plsc.store_scatter(seldst, [(nsel + k * 16 + lane) & capm], pad_dst)
    nb2 = lax.shift_right_logical(nsel - flushed + _GB - 1, 7)
    flush(nb2, flushed)
    pltpu.sync_copy(acc.at[pl.ds(0, npt)], out_hbm.at[pl.ds(lo, npt)])

  return pl.kernel(body, out_type=out_type, mesh=mesh, scratch_types=scratch,
                   compiler_params=_SC_PARAMS)


def _tc_call(fn, n, blk, args, n_out):
  grid = (n // blk,)
  in_specs = []
  for a in args:
    if a.ndim == 3:
      in_specs.append(pl.BlockSpec((2, blk, a.shape[2]),
                                   lambda i: (0, i, 0)))
    elif a.shape[0] >= n:
      in_specs.append(pl.BlockSpec((blk, a.shape[1]), lambda i: (i, 0)))
    else:
      in_specs.append(pl.BlockSpec(a.shape, lambda i: (0,) * a.ndim))
  d = args[0].shape[-1]
  out_specs = [pl.BlockSpec((blk, d), lambda i: (i, 0))] * n_out
  out_shape = [jax.ShapeDtypeStruct((n, d), jnp.float32)] * n_out
  if n_out == 1:
    out_specs, out_shape = out_specs[0], out_shape[0]
  return pl.pallas_call(fn, grid=grid, in_specs=in_specs,
                        out_specs=out_specs, out_shape=out_shape)(*args)


def _deg_of(dg_blk):
  return jnp.sum(dg_blk, axis=1, keepdims=True)


def _tc1(x_r, a_r, dg_r, w1s_r, w1n_r, b1_r, wp2_r, bp2_r, h1_r, m_r):
  deg = _deg_of(dg_r[...])
  agg = (a_r[0] + a_r[1]) / jnp.maximum(deg, 1.0)
  h1 = (jnp.dot(x_r[...], w1s_r[...], preferred_element_type=jnp.float32)
        + jnp.dot(agg, w1n_r[...], preferred_element_type=jnp.float32)
        + b1_r[...])
  h1_r[...] = h1
  m_r[...] = jnp.maximum(
      jnp.dot(h1, wp2_r[...], preferred_element_type=jnp.float32)
      + bp2_r[...], 0.0)


def _tc2(h1_r, p_r, w2s_r, w2n_r, b2_r, h2_r):
  h2_r[...] = (jnp.dot(h1_r[...], w2s_r[...],
                       preferred_element_type=jnp.float32)
               + jnp.dot(p_r[...], w2n_r[...],
                         preferred_element_type=jnp.float32)
               + b2_r[...])


def _tc3(h2_r, s_r, dg_r, w3n_r, b3_r, h3_r):
  deg = _deg_of(dg_r[...])
  hg = (s_r[0] + s_r[1] + h2_r[...]) / (deg + 1.0)
  h3_r[...] = (jnp.dot(hg, w3n_r[...], preferred_element_type=jnp.float32)
               + b3_r[...])


def _tc4(h3_r, a_r, dg_r, w4s_r, w4n_r, b4_r, wfc_r, bfc_r, o_r):
  deg = _deg_of(dg_r[...])
  agg = (a_r[0] + a_r[1]) / jnp.maximum(deg, 1.0)
  h4 = (jnp.dot(h3_r[...], w4s_r[...], preferred_element_type=jnp.float32)
        + jnp.dot(agg, w4n_r[...], preferred_element_type=jnp.float32)
        + b4_r[...])
  o_r[...] = (jnp.dot(h4, wfc_r[...], preferred_element_type=jnp.float32)
              + bfc_r[...])


def kernel(x, edge_index, W1_self, W1_neigh, b1, Wp2, bp2, W2_self, W2_neigh,
           b2, W3_neigh, b3, W4_self, W4_neigh, b4, Wfc, bfc):
  n, d = x.shape
  e = edge_index.shape[1]
  ept = e // _NW
  nblk = -(-(-(-ept // _GB)) // 16) * 16
  padt = nblk * _GB
  pad = padt - ept

  src = edge_index[0].reshape(_NW, ept)
  dst = edge_index[1].reshape(_NW, ept)
  src3 = jnp.pad(src, ((0, 0), (0, pad))).reshape(_NW, nblk, _GB)
  dst3 = jnp.pad(dst, ((0, 0), (0, pad)),
                 constant_values=n).reshape(_NW, nblk, _GB)
  src2 = src3.reshape(_NW * nblk, _GB)
  dst2 = dst3.reshape(_NW * nblk, _GB)

  npad = -(-(n + 1) // 128) * 128
  npt = -(-(n + 1) // (_NW * 8)) * 8
  zeros128 = jnp.zeros((npad, d), jnp.float32)

  b1r = b1.reshape(1, d)
  bp2r = bp2.reshape(1, d)
  b2r = b2.reshape(1, d)
  b3r = b3.reshape(1, d)
  b4r = b4.reshape(1, d)
  bfcr = bfc.reshape(1, d)

  sum_pass = _make_sc_sum(d, nblk, npad)
  deg_pass = _make_sc_deg(_NW * nblk, npt)
  seg_max = _make_sc_max(d, _NW * nblk, npt)

  blk = 2000

  degp = deg_pass(dst2).reshape(_NW * npt, 16)
  a1 = sum_pass(x, src3, dst3, zeros128)
  h1, m = _tc_call(_tc1, n, blk,
                   (x, a1, degp, W1_self, W1_neigh, b1r, Wp2, bp2r), 2)
  pooled = seg_max(m, src2, dst2, zeros128)
  h2 = _tc_call(_tc2, n, blk, (h1, pooled, W2_self, W2_neigh, b2r), 1)
  s3 = sum_pass(h2, src3, dst3, zeros128)
  h3 = _tc_call(_tc3, n, blk, (h2, s3, degp, W3_neigh, b3r), 1)
  a4 = sum_pass(h3, src3, dst3, zeros128)
  out = _tc_call(_tc4, n, blk,
                 (h3, a4, degp, W4_self, W4_neigh, b4r, Wfc, bfcr), 1)
  return out


# speedup vs baseline: 2.4765x; 1.0006x over previous
import jax
import jax.numpy as jnp
from jax import lax
from jax.experimental import pallas as pl
from jax.experimental.pallas import tpu as pltpu
from jax.experimental.pallas import tpu_sc as plsc

_NW = 32
_GB = 128
_SC_PARAMS = pltpu.CompilerParams(needs_layout_passes=False)


def _make_sc_sum(d, nblk, npad):
  mesh = plsc.VectorSubcoreMesh(core_axis_name="c", subcore_axis_name="s")
  out_type = jax.ShapeDtypeStruct((2, npad, d), jnp.float32)
  cch = 16
  nch = nblk // cch
  scratch = [
      pltpu.VMEM((cch, _GB), jnp.int32),
      pltpu.VMEM((cch, _GB), jnp.int32),
      pltpu.VMEM((2, _GB, d), jnp.float32),
      pltpu.SemaphoreType.DMA,
      pltpu.SemaphoreType.DMA,
      pltpu.SemaphoreType.DMA,
      pltpu.SemaphoreType.DMA,
      pltpu.VMEM_SHARED((npad, d), jnp.float32),
  ]
  rz = npad // 16

  def body(h_hbm, src_hbm, dst_hbm, z128_hbm,
           out_hbm, idxs, idxd, rows, g0, g1, s0, s1, acc_sh):
    cid = lax.axis_index("c")
    sid = lax.axis_index("s")
    wid = sid * 2 + cid
    pltpu.sync_copy(z128_hbm.at[pl.ds(sid * rz, rz)],
                    acc_sh.at[pl.ds(sid * rz, rz)])
    plsc.subcore_barrier()

    def chunk(ch, carry):
      pltpu.sync_copy(src_hbm.at[wid, pl.ds(ch * cch, cch)], idxs)
      pltpu.sync_copy(dst_hbm.at[wid, pl.ds(ch * cch, cch)], idxd)
      pltpu.async_copy(h_hbm.at[idxs.at[0]], rows.at[0], g0)

      def blk2(i, c2):
        b0 = i * 2
        pltpu.make_async_copy(h_hbm.at[idxs.at[b0]], rows.at[0], g0).wait()

        @pl.when(i > 0)
        def _():
          pltpu.make_async_copy(rows.at[1], acc_sh.at[idxd.at[b0 - 1]],
                                s1).wait()

        pltpu.async_copy(h_hbm.at[idxs.at[b0 + 1]], rows.at[1], g1)
        pltpu.async_copy(rows.at[0], acc_sh.at[idxd.at[b0]], s0, add=True)
        pltpu.make_async_copy(h_hbm.at[idxs.at[b0 + 1]], rows.at[1],
                              g1).wait()
        pltpu.make_async_copy(rows.at[0], acc_sh.at[idxd.at[b0]],
                              s0).wait()

        @pl.when(i + 1 < cch // 2)
        def _():
          pltpu.async_copy(h_hbm.at[idxs.at[b0 + 2]], rows.at[0], g0)

        pltpu.async_copy(rows.at[1], acc_sh.at[idxd.at[b0 + 1]], s1,
                         add=True)
        return c2

      lax.fori_loop(0, cch // 2, blk2, 0)
      pltpu.make_async_copy(rows.at[1], acc_sh.at[idxd.at[cch - 1]],
                            s1).wait()
      return carry

    lax.fori_loop(0, nch, chunk, 0)
    plsc.subcore_barrier()
    pltpu.sync_copy(acc_sh.at[pl.ds(sid * rz, rz)],
                    out_hbm.at[cid, pl.ds(sid * rz, rz)])

  return pl.kernel(body, out_type=out_type, mesh=mesh, scratch_types=scratch,
                   compiler_params=_SC_PARAMS)


def _make_sc_deg(nblk_tot, npt):
  mesh = plsc.VectorSubcoreMesh(core_axis_name="c", subcore_axis_name="s")
  cch = 32
  nch = nblk_tot // cch
  drows = npt // 8

  out_type = jax.ShapeDtypeStruct((_NW * drows, 128), jnp.float32)
  scratch = [
      pltpu.VMEM((2, cch, _GB), jnp.int32),
      pltpu.VMEM((drows, 128), jnp.float32),
      pltpu.SemaphoreType.DMA,
  ]

  def body(dst2_hbm, out_hbm, dbuf, dacc, st):
    cid = lax.axis_index("c")
    sid = lax.axis_index("s")
    wid = sid * 2 + cid
    lo = wid * npt
    zeros = jnp.zeros((16,), jnp.float32)

    def zrow(r, carry):
      for c in range(8):
        dacc[r, pl.ds(c * 16, 16)] = zeros
      return carry

    lax.fori_loop(0, drows, zrow, 0)

    lane = lax.iota(jnp.int32, 16)
    onesv = jnp.ones((16,), jnp.float32)
    pltpu.sync_copy(dst2_hbm.at[pl.ds(0, cch)], dbuf.at[0])

    def chunk(ch, carry):
      slot = ch % 2
      nxt = (ch + 1) % 2

      @pl.when(ch + 1 < nch)
      def _():
        pltpu.async_copy(dst2_hbm.at[pl.ds((ch + 1) * cch, cch)],
                         dbuf.at[nxt], st)

      def row(r, c2):
        for c in range(_GB // 16):
          dv = dbuf[slot, r, pl.ds(c * 16, 16)] - lo
          msk = (dv >= 0) & (dv < npt)
          rr = lax.shift_right_logical(dv, 3)
          cc = lax.shift_left(dv & 7, 4) + lane
          plsc.addupdate_scatter(dacc, [rr, cc], onesv, mask=msk)
        return c2

      lax.fori_loop(0, cch, row, 0)

      @pl.when(ch + 1 < nch)
      def _():
        pltpu.make_async_copy(dst2_hbm.at[pl.ds((ch + 1) * cch, cch)],
                              dbuf.at[nxt], st).wait()

      return carry

    lax.fori_loop(0, nch, chunk, 0)
    pltpu.sync_copy(dacc, out_hbm.at[pl.ds(wid * drows, drows)])

  return pl.kernel(body, out_type=out_type, mesh=mesh, scratch_types=scratch,
                   compiler_params=_SC_PARAMS)


def _make_sc_max(d, nblk_tot, npt):
  mesh = plsc.VectorSubcoreMesh(core_axis_name="c", subcore_axis_name="s")
  cch = 32
  nch = nblk_tot // cch
  cap = 8192
  acc_rows = npt + 8

  out_type = jax.ShapeDtypeStruct((_NW * npt, d), jnp.float32)
  scratch = [
      pltpu.VMEM((2, cch, _GB), jnp.int32),
      pltpu.VMEM((2, cch, _GB), jnp.int32),
      pltpu.VMEM((cap,), jnp.int32),
      pltpu.VMEM((cap,), jnp.int32),
      pltpu.VMEM((2, _GB, d), jnp.float32),
      pltpu.VMEM((acc_rows, d), jnp.float32),
      pltpu.SemaphoreType.DMA,
      pltpu.SemaphoreType.DMA,
      pltpu.SemaphoreType.DMA,
  ]

  def body(m_hbm, src2_hbm, dst2_hbm, z128_hbm,
           out_hbm, sbuf, dbuf, selsrc, seldst, rows2, acc, st, g0, g1):
    cid = lax.axis_index("c")
    sid = lax.axis_index("s")
    wid = sid * 2 + cid
    lo = wid * npt
    pltpu.sync_copy(z128_hbm.at[pl.ds(0, acc_rows)], acc)

    lane = lax.iota(jnp.int32, 16)
    pad_src = jnp.zeros((16,), jnp.int32)
    pad_dst = jnp.full((16,), npt, jnp.int32)
    capm = cap - 1

    gsem = (g0, g1)

    def flush(nb, flushed):
      def boff(b):
        return pl.multiple_of((flushed + b * _GB) & capm, _GB)

      @pl.when(nb > 0)
      def _():
        pltpu.async_copy(m_hbm.at[selsrc.at[pl.ds(boff(0), _GB)]],
                         rows2.at[0], g0)

      def gblk(b, c2):
        off = boff(b)

        def proc(par):
          pltpu.make_async_copy(m_hbm.at[selsrc.at[pl.ds(off, _GB)]],
                                rows2.at[par], gsem[par]).wait()

          @pl.when(b + 1 < nb)
          def _():
            pltpu.async_copy(m_hbm.at[selsrc.at[pl.ds(boff(b + 1), _GB)]],
                             rows2.at[1 - par], gsem[1 - par])

          def grp(k, c3):
            dvec = seldst[pl.ds(pl.multiple_of(off + k * 16, 16), 16)]
            for j in range(16):
              dl = dvec[j]
              for c in range(d // 16):
                cur = acc[dl, pl.ds(c * 16, 16)]
                val = rows2[par, k * 16 + j, pl.ds(c * 16, 16)]
                acc[dl, pl.ds(c * 16, 16)] = jnp.maximum(cur, val)
            return c3

          lax.fori_loop(0, _GB // 16, grp, 0)

        @pl.when(b % 2 == 0)
        def _():
          proc(0)

        @pl.when(b % 2 == 1)
        def _():
          proc(1)

        return c2

      lax.fori_loop(0, nb, gblk, 0)
      return flushed + nb * _GB

    pltpu.sync_copy(src2_hbm.at[pl.ds(0, cch)], sbuf.at[0])
    pltpu.sync_copy(dst2_hbm.at[pl.ds(0, cch)], dbuf.at[0])

    def chunk(ch, carry):
      nsel, flushed = carry
      slot = ch % 2
      nxt = (ch + 1) % 2

      @pl.when(ch + 1 < nch)
      def _():
        pltpu.async_copy(src2_hbm.at[pl.ds((ch + 1) * cch, cch)],
                         sbuf.at[nxt], st)
        pltpu.async_copy(dst2_hbm.at[pl.ds((ch + 1) * cch, cch)],
                         dbuf.at[nxt], st)

      def row(r, nsel):
        for c in range(_GB // 16):
          dv = dbuf[slot, r, pl.ds(c * 16, 16)]
          sv = sbuf[slot, r, pl.ds(c * 16, 16)]
          msk = (dv >= lo) & (dv < lo + npt)
          inc = plsc.cumsum(msk.astype(jnp.int32))
          pos = (nsel + inc - 1) & capm
          plsc.store_scatter(selsrc, [pos], sv, mask=msk)
          plsc.store_scatter(seldst, [pos], dv - lo, mask=msk)
          nsel = nsel + inc[15]
        return nsel

      nsel = lax.fori_loop(0, cch, row, nsel)
      nbavail = lax.shift_right_logical(nsel - flushed, 7)
      flushed = flush(nbavail - (nbavail & 1), flushed)

      @pl.when(ch + 1 < nch)
      def _():
        pltpu.make_async_copy(src2_hbm.at[pl.ds((ch + 1) * cch, cch)],
                              sbuf.at[nxt], st).wait()
        pltpu.make_async_copy(dst2_hbm.at[pl.ds((ch + 1) * cch, cch)],
                              dbuf.at[nxt], st).wait()

      return (nsel, flushed)

    nsel, flushed = lax.fori_loop(0, nch, chunk,
                                  (jnp.int32(0), jnp.int32(0)))
    for k in range(_GB // 16):
      plsc.store_scatter(selsrc, [(nsel + k * 16 + lane) & capm], pad_src)
      plsc.store_scatter(seldst, [(nsel + k * 16 + lane) & capm], pad_dst)
    nb2 = lax.shift_right_logical(nsel - flushed + _GB - 1, 7)
    flush(nb2, flushed)
    pltpu.sync_copy(acc.at[pl.ds(0, npt)], out_hbm.at[pl.ds(lo, npt)])

  return pl.kernel(body, out_type=out_type, mesh=mesh, scratch_types=scratch,
                   compiler_params=_SC_PARAMS)


def _tc_call(fn, n, blk, args, n_out):
  grid = (n // blk,)
  in_specs = []
  for a in args:
    if a.ndim == 3:
      in_specs.append(pl.BlockSpec((2, blk, a.shape[2]),
                                   lambda i: (0, i, 0)))
    elif a.shape[0] >= n:
      in_specs.append(pl.BlockSpec((blk, a.shape[1]), lambda i: (i, 0)))
    else:
      in_specs.append(pl.BlockSpec(a.shape, lambda i: (0,) * a.ndim))
  d = args[0].shape[-1]
  out_specs = [pl.BlockSpec((blk, d), lambda i: (i, 0))] * n_out
  out_shape = [jax.ShapeDtypeStruct((n, d), jnp.float32)] * n_out
  if n_out == 1:
    out_specs, out_shape = out_specs[0], out_shape[0]
  return pl.pallas_call(fn, grid=grid, in_specs=in_specs,
                        out_specs=out_specs, out_shape=out_shape)(*args)


def _deg_of(dg_blk):
  return jnp.sum(dg_blk, axis=1, keepdims=True)


def _tc1(x_r, a_r, dg_r, w1s_r, w1n_r, b1_r, wp2_r, bp2_r, h1_r, m_r):
  deg = _deg_of(dg_r[...])
  agg = (a_r[0] + a_r[1]) / jnp.maximum(deg, 1.0)
  h1 = (jnp.dot(x_r[...], w1s_r[...], preferred_element_type=jnp.float32)
        + jnp.dot(agg, w1n_r[...], preferred_element_type=jnp.float32)
        + b1_r[...])
  h1_r[...] = h1
  m_r[...] = jnp.maximum(
      jnp.dot(h1, wp2_r[...], preferred_element_type=jnp.float32)
      + bp2_r[...], 0.0)


def _tc2(h1_r, p_r, w2s_r, w2n_r, b2_r, h2_r):
  h2_r[...] = (jnp.dot(h1_r[...], w2s_r[...],
                       preferred_element_type=jnp.float32)
               + jnp.dot(p_r[...], w2n_r[...],
                         preferred_element_type=jnp.float32)
               + b2_r[...])


def _tc3(h2_r, s_r, dg_r, w3n_r, b3_r, h3_r):
  deg = _deg_of(dg_r[...])
  hg = (s_r[0] + s_r[1] + h2_r[...]) / (deg + 1.0)
  h3_r[...] = (jnp.dot(hg, w3n_r[...], preferred_element_type=jnp.float32)
               + b3_r[...])


def _tc4(h3_r, a_r, dg_r, w4s_r, w4n_r, b4_r, wfc_r, bfc_r, o_r):
  deg = _deg_of(dg_r[...])
  agg = (a_r[0] + a_r[1]) / jnp.maximum(deg, 1.0)
  h4 = (jnp.dot(h3_r[...], w4s_r[...], preferred_element_type=jnp.float32)
        + jnp.dot(agg, w4n_r[...], preferred_element_type=jnp.float32)
        + b4_r[...])
  o_r[...] = (jnp.dot(h4, wfc_r[...], preferred_element_type=jnp.float32)
              + bfc_r[...])


def kernel(x, edge_index, W1_self, W1_neigh, b1, Wp2, bp2, W2_self, W2_neigh,
           b2, W3_neigh, b3, W4_self, W4_neigh, b4, Wfc, bfc):
  n, d = x.shape
  e = edge_index.shape[1]
  ept = e // _NW
  nblk = -(-(-(-ept // _GB)) // 16) * 16
  padt = nblk * _GB
  pad = padt - ept

  src = edge_index[0].reshape(_NW, ept)
  dst = edge_index[1].reshape(_NW, ept)
  src3 = jnp.pad(src, ((0, 0), (0, pad))).reshape(_NW, nblk, _GB)
  dst3 = jnp.pad(dst, ((0, 0), (0, pad)),
                 constant_values=n).reshape(_NW, nblk, _GB)
  src2 = src3.reshape(_NW * nblk, _GB)
  dst2 = dst3.reshape(_NW * nblk, _GB)

  npad = -(-(n + 1) // 128) * 128
  npt = -(-(n + 1) // (_NW * 8)) * 8
  zeros128 = jnp.zeros((npad, d), jnp.float32)

  b1r = b1.reshape(1, d)
  bp2r = bp2.reshape(1, d)
  b2r = b2.reshape(1, d)
  b3r = b3.reshape(1, d)
  b4r = b4.reshape(1, d)
  bfcr = bfc.reshape(1, d)

  sum_pass = _make_sc_sum(d, nblk, npad)
  deg_pass = _make_sc_deg(_NW * nblk, npt)
  seg_max = _make_sc_max(d, _NW * nblk, npt)

  blk = 2000

  degp = deg_pass(dst2).reshape(_NW * npt, 16)
  a1 = sum_pass(x, src3, dst3, zeros128)
  h1, m = _tc_call(_tc1, n, blk,
                   (x, a1, degp, W1_self, W1_neigh, b1r, Wp2, bp2r), 2)
  pooled = seg_max(m, src2, dst2, zeros128)
  h2 = _tc_call(_tc2, n, blk, (h1, pooled, W2_self, W2_neigh, b2r), 1)
  s3 = sum_pass(h2, src3, dst3, zeros128)
  h3 = _tc_call(_tc3, n, blk, (h2, s3, degp, W3_neigh, b3r), 1)
  a4 = sum_pass(h3, src3, dst3, zeros128)
  out = _tc_call(_tc4, n, blk,
                 (h3, a4, degp, W4_self, W4_neigh, b4r, Wfc, bfcr), 1)
  return out
